# baseline (speedup 1.0000x reference)
import functools

import jax
import jax.numpy as jnp
from jax import lax
from jax.experimental import pallas as pl
from jax.experimental.pallas import tpu as pltpu
from jax.experimental.pallas import tpu_sc as plsc

NUM_WORKERS = 32
LANES = 16


def _make_sc_embed(batch, num_cards, out_dim, num_keys):
    assert batch % (NUM_WORKERS * LANES) == 0
    per_worker = batch // NUM_WORKERS
    mesh = plsc.VectorSubcoreMesh(core_axis_name="c", subcore_axis_name="s")

    f_tiles = out_dim // 8
    groups = batch // 128
    tile_words = 8 * 128
    run = (per_worker // 128) * tile_words

    @functools.partial(
        pl.kernel,
        mesh=mesh,
        compiler_params=pltpu.CompilerParams(
            use_tc_tiling_on_sc=False, needs_layout_passes=False),
        out_type=jax.ShapeDtypeStruct(
            (num_cards, f_tiles, groups * tile_words), jnp.float32),
        scratch_types=[
            pltpu.VMEM((num_keys * out_dim,), jnp.float32),
            pltpu.VMEM((num_cards, per_worker), jnp.int32),
            [pltpu.VMEM((f_tiles, run), jnp.float32)] * 2,
            [pltpu.SemaphoreType.DMA] * 2,
        ],
    )
    def sc_embed(keys_hbm, table_hbm, out_hbm, tab_v, keys_v, block_v, sem_s):
        wid = lax.axis_index("s") * 2 + lax.axis_index("c")
        b0 = wid * per_worker

        pltpu.sync_copy(table_hbm, tab_v)
        pltpu.sync_copy(keys_hbm.at[:, pl.ds(b0, per_worker)], keys_v)

        def do_card(c, bb, first):
            @pl.when(jnp.logical_not(first))
            def _():
                pltpu.make_async_copy(
                    block_v[bb],
                    out_hbm.at[c, :, pl.ds(wid * run, run)],
                    sem_s[bb],
                ).wait()

            @plsc.parallel_loop(0, per_worker // LANES, unroll=4)
            def _fill(i):
                kvec = keys_v[c, pl.ds(i * LANES, LANES)]
                ib = (i // 8) * tile_words + (i % 8) * LANES
                vals = [plsc.load_gather(tab_v, [kvec + (f * num_keys)])
                        for f in range(out_dim)]
                for f in range(out_dim):
                    block_v[bb][f // 8, pl.ds(ib + (f % 8) * 128, LANES)] = (
                        vals[f])

            pltpu.async_copy(
                block_v[bb],
                out_hbm.at[c, :, pl.ds(wid * run, run)],
                sem_s[bb],
            )

        def pair_body(t, carry):
            do_card(2 * t, 0, t == 0)
            do_card(2 * t + 1, 1, t == 0)
            return carry

        lax.fori_loop(0, num_cards // 2, pair_body, 0)
        for bb in range(2):
            pltpu.make_async_copy(
                block_v[bb],
                out_hbm.at[0, :, pl.ds(wid * run, run)],
                sem_s[bb],
            ).wait()

    return sc_embed


_sc_embed_cached = None


def _get_sc_embed(batch, num_cards, out_dim, num_keys):
    global _sc_embed_cached
    if _sc_embed_cached is None:
        _sc_embed_cached = _make_sc_embed(batch, num_cards, out_dim, num_keys)
    return _sc_embed_cached


def kernel(ranks, suits, rank_weight, suit_weight):
    batch, num_cards = ranks.shape
    num_ranks, rank_dim = rank_weight.shape
    num_suits, suit_dim = suit_weight.shape
    out_dim = rank_dim + suit_dim
    num_keys = num_ranks * num_suits

    combined = jnp.concatenate(
        [
            jnp.repeat(rank_weight, num_suits, axis=0),
            jnp.tile(suit_weight, (num_ranks, 1)),
        ],
        axis=1,
    )
    table_t = combined.T.reshape(-1)

    keys_t = (ranks * num_suits + suits).T

    fn = _get_sc_embed(batch, num_cards, out_dim, num_keys)
    out_tiled = fn(keys_t, table_t)
    out5 = out_tiled.reshape(num_cards, out_dim // 8, batch // 128, 8, 128)
    out = jnp.transpose(out5, (2, 4, 0, 1, 3))
    return out.reshape(batch, num_cards, out_dim)


# speedup vs baseline: 1.8168x; 1.8168x over previous
import functools

import jax
import jax.numpy as jnp
from jax import lax
from jax.experimental import pallas as pl
from jax.experimental.pallas import tpu as pltpu
from jax.experimental.pallas import tpu_sc as plsc

NUM_WORKERS = 32
LANES = 16


def _make_sc_embed(batch, num_cards, out_dim, num_keys):
    assert batch % (NUM_WORKERS * LANES) == 0
    per_worker = batch // NUM_WORKERS
    mesh = plsc.VectorSubcoreMesh(core_axis_name="c", subcore_axis_name="s")

    f_tiles = out_dim // 8
    groups = batch // 128
    tile_words = 8 * 128
    run = (per_worker // 128) * tile_words

    @functools.partial(
        pl.kernel,
        mesh=mesh,
        compiler_params=pltpu.CompilerParams(
            use_tc_tiling_on_sc=False, needs_layout_passes=False),
        out_type=jax.ShapeDtypeStruct(
            (num_cards, f_tiles, groups * tile_words), jnp.float32),
        scratch_types=[
            pltpu.VMEM((num_keys * out_dim,), jnp.float32),
            pltpu.VMEM((num_cards, per_worker), jnp.int32),
            [pltpu.VMEM((f_tiles, run), jnp.float32)] * 2,
            [pltpu.SemaphoreType.DMA] * 2,
        ],
    )
    def sc_embed(keys_hbm, table_hbm, out_hbm, tab_v, keys_v, block_v, sem_s):
        wid = lax.axis_index("s") * 2 + lax.axis_index("c")
        b0 = wid * per_worker

        pltpu.sync_copy(table_hbm, tab_v)
        pltpu.sync_copy(keys_hbm.at[:, pl.ds(b0, per_worker)], keys_v)

        def do_card(c, bb, first):
            @pl.when(jnp.logical_not(first))
            def _():
                pltpu.make_async_copy(
                    block_v[bb],
                    out_hbm.at[c, :, pl.ds(wid * run, run)],
                    sem_s[bb],
                ).wait()

            @plsc.parallel_loop(0, per_worker // LANES, unroll=1)
            def _fill(i):
                kvec = keys_v[c, pl.ds(i * LANES, LANES)]
                ib = (i // 8) * tile_words + (i % 8) * LANES
                vals = [plsc.load_gather(tab_v, [kvec + (f * num_keys)])
                        for f in range(out_dim)]
                for f in range(out_dim):
                    block_v[bb][f // 8, pl.ds(ib + (f % 8) * 128, LANES)] = (
                        vals[f])

            pltpu.async_copy(
                block_v[bb],
                out_hbm.at[c, :, pl.ds(wid * run, run)],
                sem_s[bb],
            )

        def pair_body(t, carry):
            do_card(2 * t, 0, t == 0)
            do_card(2 * t + 1, 1, t == 0)
            return carry

        lax.fori_loop(0, num_cards // 2, pair_body, 0)
        for bb in range(2):
            pltpu.make_async_copy(
                block_v[bb],
                out_hbm.at[0, :, pl.ds(wid * run, run)],
                sem_s[bb],
            ).wait()

    return sc_embed


_sc_embed_cached = None


def _get_sc_embed(batch, num_cards, out_dim, num_keys):
    global _sc_embed_cached
    if _sc_embed_cached is None:
        _sc_embed_cached = _make_sc_embed(batch, num_cards, out_dim, num_keys)
    return _sc_embed_cached


def kernel(ranks, suits, rank_weight, suit_weight):
    batch, num_cards = ranks.shape
    num_ranks, rank_dim = rank_weight.shape
    num_suits, suit_dim = suit_weight.shape
    out_dim = rank_dim + suit_dim
    num_keys = num_ranks * num_suits

    combined = jnp.concatenate(
        [
            jnp.repeat(rank_weight, num_suits, axis=0),
            jnp.tile(suit_weight, (num_ranks, 1)),
        ],
        axis=1,
    )
    table_t = combined.T.reshape(-1)

    keys_t = (ranks * num_suits + suits).T

    fn = _get_sc_embed(batch, num_cards, out_dim, num_keys)
    out_tiled = fn(keys_t, table_t)
    out5 = out_tiled.reshape(num_cards, out_dim // 8, batch // 128, 8, 128)
    out = jnp.transpose(out5, (2, 4, 0, 1, 3))
    return out.reshape(batch, num_cards, out_dim)


# speedup vs baseline: 1.8419x; 1.0138x over previous
import functools

import jax
import jax.numpy as jnp
from jax import lax
from jax.experimental import pallas as pl
from jax.experimental.pallas import tpu as pltpu
from jax.experimental.pallas import tpu_sc as plsc

NUM_WORKERS = 32
LANES = 16


def _make_sc_embed(batch, num_cards, out_dim, num_keys):
    assert batch % (NUM_WORKERS * LANES) == 0
    per_worker = batch // NUM_WORKERS
    mesh = plsc.VectorSubcoreMesh(core_axis_name="c", subcore_axis_name="s")

    f_tiles = out_dim // 8
    groups = batch // 128
    tile_words = 8 * 128
    run = (per_worker // 128) * tile_words

    @functools.partial(
        pl.kernel,
        mesh=mesh,
        compiler_params=pltpu.CompilerParams(
            use_tc_tiling_on_sc=False, needs_layout_passes=False),
        out_type=jax.ShapeDtypeStruct(
            (num_cards, f_tiles, groups * tile_words), jnp.float32),
        scratch_types=[
            pltpu.VMEM((num_keys * out_dim,), jnp.float32),
            pltpu.VMEM((num_cards, per_worker), jnp.int32),
            [pltpu.VMEM((f_tiles, run), jnp.float32)] * 2,
            [pltpu.SemaphoreType.DMA] * 2,
        ],
    )
    def sc_embed(keys_hbm, table_hbm, out_hbm, tab_v, keys_v, block_v, sem_s):
        wid = lax.axis_index("s") * 2 + lax.axis_index("c")
        b0 = wid * per_worker

        pltpu.sync_copy(table_hbm, tab_v)
        pltpu.sync_copy(keys_hbm.at[:, pl.ds(b0, per_worker)], keys_v)

        def do_card(c, bb, first):
            @pl.when(jnp.logical_not(first))
            def _():
                pltpu.make_async_copy(
                    block_v[bb],
                    out_hbm.at[c, :, pl.ds(wid * run, run)],
                    sem_s[bb],
                ).wait()

            @plsc.parallel_loop(0, per_worker // LANES, unroll=2)
            def _fill(i):
                kvec = keys_v[c, pl.ds(i * LANES, LANES)]
                ib = (i // 8) * tile_words + (i % 8) * LANES
                vals = [plsc.load_gather(tab_v, [kvec + (f * num_keys)])
                        for f in range(out_dim)]
                for f in range(out_dim):
                    block_v[bb][f // 8, pl.ds(ib + (f % 8) * 128, LANES)] = (
                        vals[f])

            pltpu.async_copy(
                block_v[bb],
                out_hbm.at[c, :, pl.ds(wid * run, run)],
                sem_s[bb],
            )

        def pair_body(t, carry):
            do_card(2 * t, 0, t == 0)
            do_card(2 * t + 1, 1, t == 0)
            return carry

        lax.fori_loop(0, num_cards // 2, pair_body, 0)
        for bb in range(2):
            pltpu.make_async_copy(
                block_v[bb],
                out_hbm.at[0, :, pl.ds(wid * run, run)],
                sem_s[bb],
            ).wait()

    return sc_embed


_sc_embed_cached = None


def _get_sc_embed(batch, num_cards, out_dim, num_keys):
    global _sc_embed_cached
    if _sc_embed_cached is None:
        _sc_embed_cached = _make_sc_embed(batch, num_cards, out_dim, num_keys)
    return _sc_embed_cached


def kernel(ranks, suits, rank_weight, suit_weight):
    batch, num_cards = ranks.shape
    num_ranks, rank_dim = rank_weight.shape
    num_suits, suit_dim = suit_weight.shape
    out_dim = rank_dim + suit_dim
    num_keys = num_ranks * num_suits

    combined = jnp.concatenate(
        [
            jnp.repeat(rank_weight, num_suits, axis=0),
            jnp.tile(suit_weight, (num_ranks, 1)),
        ],
        axis=1,
    )
    table_t = combined.T.reshape(-1)

    keys_t = (ranks * num_suits + suits).T

    fn = _get_sc_embed(batch, num_cards, out_dim, num_keys)
    out_tiled = fn(keys_t, table_t)
    out5 = out_tiled.reshape(num_cards, out_dim // 8, batch // 128, 8, 128)
    out = jnp.transpose(out5, (2, 4, 0, 1, 3))
    return out.reshape(batch, num_cards, out_dim)
